# baseline (device time: 26012 ns/iter reference)
import jax
import jax.numpy as jnp
from jax import lax
from jax.experimental import pallas as pl
from jax.experimental.pallas import tpu as pltpu

N_DEV = 4
B, SQ, SKV, DH = 2, 256, 256, 64
HQ_TOTAL = 16
HQ_LOC = HQ_TOTAL // N_DEV
D_MODEL = 512
BLK = 64
NH = B * HQ_LOC


def kernel(x, Wq, K_ext, V_ext, Wo):

    def body(x_ref, wq_ref, k_hbm, v_hbm, wo_ref, out_ref,
             k_sc, v_sc, comm_ref, send_ref, copy_sems, send_sems, recv_sems):
        my_pos = lax.axis_index("i")
        p1 = my_pos ^ 1
        p2 = (N_DEV - 1) - my_pos

        copies = []
        for b in range(B):
            for h in range(HQ_LOC):
                i = b * HQ_LOC + h
                hg = my_pos * HQ_LOC + h
                ck = pltpu.make_async_copy(
                    k_hbm.at[b, :, hg, :], k_sc.at[i], copy_sems.at[i])
                cv = pltpu.make_async_copy(
                    v_hbm.at[b, :, hg, :], v_sc.at[i], copy_sems.at[NH + i])
                ck.start()
                cv.start()
                copies.append((ck, cv))

        barrier_sem = pltpu.get_barrier_semaphore()
        for nbr in (p1, p2):
            pl.semaphore_signal(
                barrier_sem, inc=1,
                device_id=(nbr,), device_id_type=pl.DeviceIdType.MESH,
            )
        pl.semaphore_wait(barrier_sem, 2)

        def exchange(slot, partner):
            return pltpu.make_async_remote_copy(
                src_ref=send_ref.at[slot],
                dst_ref=comm_ref.at[slot],
                send_sem=send_sems.at[slot],
                recv_sem=recv_sems.at[slot],
                device_id=(partner,),
                device_id_type=pl.DeviceIdType.MESH,
            )

        x2 = x_ref[...].reshape(B * SQ, D_MODEL).astype(jnp.bfloat16)
        wq = wq_ref[...].astype(jnp.bfloat16)
        wo = wo_ref[...].astype(jnp.bfloat16)
        q = jnp.dot(x2, wq, preferred_element_type=jnp.float32)

        qb = lax.broadcasted_iota(jnp.int32, (SQ, SKV), 0) // BLK
        kb = lax.broadcasted_iota(jnp.int32, (SQ, SKV), 1) // BLK
        mask = kb <= qb

        def batch_partial(b):
            parts = []
            for h in range(HQ_LOC):
                i = b * HQ_LOC + h
                ck, cv = copies[i]
                ck.wait()
                cv.wait()
                qbh = q[b * SQ:(b + 1) * SQ, h * DH:(h + 1) * DH]
                qbh = qbh.astype(jnp.bfloat16)
                kbh = k_sc[i].astype(jnp.bfloat16)
                s = lax.dot_general(
                    qbh, kbh, (((1,), (1,)), ((), ())),
                    preferred_element_type=jnp.float32,
                ) * 0.125
                w = jnp.exp(jnp.where(mask, s, -1e9))
                w = w / jnp.sum(w, axis=1, keepdims=True)
                ctx = jnp.dot(w.astype(jnp.bfloat16),
                              v_sc[i].astype(jnp.bfloat16),
                              preferred_element_type=jnp.float32)
                parts.append(ctx.astype(jnp.bfloat16))
            ctx_b = jnp.concatenate(parts, axis=1)
            return jnp.dot(ctx_b, wo,
                           preferred_element_type=jnp.float32)

        pA = batch_partial(0)
        send_ref[0] = pA.astype(jnp.bfloat16)
        ex0 = exchange(0, p1)
        ex0.start()

        pB = batch_partial(1)
        send_ref[1] = pB.astype(jnp.bfloat16)
        ex1 = exchange(1, p2)
        ex1.start()

        ex0.wait()
        accA = pA + comm_ref[0].astype(jnp.float32)
        send_ref[2] = accA.astype(jnp.bfloat16)
        ex2 = exchange(2, p2)
        ex2.start()

        ex1.wait()
        accB = pB + comm_ref[1].astype(jnp.float32)
        send_ref[3] = accB.astype(jnp.bfloat16)
        ex3 = exchange(3, p1)
        ex3.start()

        ex2.wait()
        out_ref[0] = accA + comm_ref[2].astype(jnp.float32)
        ex3.wait()
        out_ref[1] = accB + comm_ref[3].astype(jnp.float32)

    return pl.pallas_call(
        body,
        out_shape=jax.ShapeDtypeStruct((B, SQ, D_MODEL), jnp.float32),
        in_specs=[
            pl.BlockSpec(memory_space=pltpu.VMEM),
            pl.BlockSpec(memory_space=pltpu.VMEM),
            pl.BlockSpec(memory_space=pltpu.MemorySpace.HBM),
            pl.BlockSpec(memory_space=pltpu.MemorySpace.HBM),
            pl.BlockSpec(memory_space=pltpu.VMEM),
        ],
        out_specs=pl.BlockSpec(memory_space=pltpu.VMEM),
        scratch_shapes=[
            pltpu.VMEM((NH, SKV, DH), jnp.float32),
            pltpu.VMEM((NH, SKV, DH), jnp.float32),
            pltpu.VMEM((4, SQ, D_MODEL), jnp.bfloat16),
            pltpu.VMEM((4, SQ, D_MODEL), jnp.bfloat16),
            pltpu.SemaphoreType.DMA((2 * NH,)),
            pltpu.SemaphoreType.DMA((4,)),
            pltpu.SemaphoreType.DMA((4,)),
        ],
        compiler_params=pltpu.CompilerParams(collective_id=0),
    )(x, Wq, K_ext, V_ext, Wo)


# device time: 17267 ns/iter; 1.5065x vs baseline; 1.5065x over previous
import jax
import jax.numpy as jnp
from jax import lax
from jax.experimental import pallas as pl
from jax.experimental.pallas import tpu as pltpu

N_DEV = 4
B, SQ, SKV, DH = 2, 256, 256, 64
HQ_TOTAL = 16
HQ_LOC = HQ_TOTAL // N_DEV
D_MODEL = 512
BLK = 64
NH = B * HQ_LOC


def kernel(x, Wq, K_ext, V_ext, Wo):

    my = lax.axis_index("i")

    def pack(t):
        t = lax.dynamic_slice_in_dim(t, my * HQ_LOC, HQ_LOC, axis=2)
        t = t.astype(jnp.bfloat16)
        return t.transpose(0, 2, 1, 3).reshape(NH, SKV, DH)

    k_loc = pack(K_ext)
    v_loc = pack(V_ext)

    def body(x_ref, wq_ref, k_ref, v_ref, wo_ref, out_ref,
             comm_ref, send_ref, send_sems, recv_sems):
        my_pos = lax.axis_index("i")
        p1 = my_pos ^ 1
        p2 = (N_DEV - 1) - my_pos

        barrier_sem = pltpu.get_barrier_semaphore()
        for nbr in (p1, p2):
            pl.semaphore_signal(
                barrier_sem, inc=1,
                device_id=(nbr,), device_id_type=pl.DeviceIdType.MESH,
            )
        pl.semaphore_wait(barrier_sem, 2)

        def exchange(slot, partner):
            return pltpu.make_async_remote_copy(
                src_ref=send_ref.at[slot],
                dst_ref=comm_ref.at[slot],
                send_sem=send_sems.at[slot],
                recv_sem=recv_sems.at[slot],
                device_id=(partner,),
                device_id_type=pl.DeviceIdType.MESH,
            )

        x2 = x_ref[...].reshape(B * SQ, D_MODEL).astype(jnp.bfloat16)
        wq = wq_ref[...].astype(jnp.bfloat16)
        wo = wo_ref[...].astype(jnp.bfloat16)
        q = jnp.dot(x2, wq, preferred_element_type=jnp.float32)

        qb = lax.broadcasted_iota(jnp.int32, (SQ, SKV), 0) // BLK
        kb = lax.broadcasted_iota(jnp.int32, (SQ, SKV), 1) // BLK
        mask = kb <= qb

        def batch_partial(b):
            parts = []
            for h in range(HQ_LOC):
                i = b * HQ_LOC + h
                qbh = q[b * SQ:(b + 1) * SQ, h * DH:(h + 1) * DH]
                qbh = qbh.astype(jnp.bfloat16)
                kbh = k_ref[i]
                s = lax.dot_general(
                    qbh, kbh, (((1,), (1,)), ((), ())),
                    preferred_element_type=jnp.float32,
                ) * 0.125
                w = jnp.exp(jnp.where(mask, s, -1e9))
                w = w / jnp.sum(w, axis=1, keepdims=True)
                ctx = jnp.dot(w.astype(jnp.bfloat16), v_ref[i],
                              preferred_element_type=jnp.float32)
                parts.append(ctx.astype(jnp.bfloat16))
            ctx_b = jnp.concatenate(parts, axis=1)
            return jnp.dot(ctx_b, wo,
                           preferred_element_type=jnp.float32)

        pA = batch_partial(0)
        send_ref[0] = pA.astype(jnp.bfloat16)
        ex0 = exchange(0, p1)
        ex0.start()

        pB = batch_partial(1)
        send_ref[1] = pB.astype(jnp.bfloat16)
        ex1 = exchange(1, p2)
        ex1.start()

        ex0.wait()
        accA = pA + comm_ref[0].astype(jnp.float32)
        send_ref[2] = accA.astype(jnp.bfloat16)
        ex2 = exchange(2, p2)
        ex2.start()

        ex1.wait()
        accB = pB + comm_ref[1].astype(jnp.float32)
        send_ref[3] = accB.astype(jnp.bfloat16)
        ex3 = exchange(3, p1)
        ex3.start()

        ex2.wait()
        out_ref[0] = accA + comm_ref[2].astype(jnp.float32)
        ex3.wait()
        out_ref[1] = accB + comm_ref[3].astype(jnp.float32)

    return pl.pallas_call(
        body,
        out_shape=jax.ShapeDtypeStruct((B, SQ, D_MODEL), jnp.float32),
        in_specs=[pl.BlockSpec(memory_space=pltpu.VMEM)] * 5,
        out_specs=pl.BlockSpec(memory_space=pltpu.VMEM),
        scratch_shapes=[
            pltpu.VMEM((4, SQ, D_MODEL), jnp.bfloat16),
            pltpu.VMEM((4, SQ, D_MODEL), jnp.bfloat16),
            pltpu.SemaphoreType.DMA((4,)),
            pltpu.SemaphoreType.DMA((4,)),
        ],
        compiler_params=pltpu.CompilerParams(collective_id=0),
    )(x, Wq, k_loc, v_loc, Wo)


# device time: 17061 ns/iter; 1.5246x vs baseline; 1.0121x over previous
import jax
import jax.numpy as jnp
from jax import lax
from jax.experimental import pallas as pl
from jax.experimental.pallas import tpu as pltpu

N_DEV = 4
B, SQ, SKV, DH = 2, 256, 256, 64
HQ_TOTAL = 16
HQ_LOC = HQ_TOTAL // N_DEV
D_MODEL = 512
BLK = 64
NH = B * HQ_LOC


def kernel(x, Wq, K_ext, V_ext, Wo):

    my = lax.axis_index("i")

    def pack(t):
        t = t.reshape(B, SKV, HQ_TOTAL * DH)
        t = lax.dynamic_slice_in_dim(t, my * HQ_LOC * DH, HQ_LOC * DH, axis=2)
        return t.astype(jnp.bfloat16)

    k_loc = pack(K_ext)
    v_loc = pack(V_ext)

    def body(x_ref, wq_ref, k_ref, v_ref, wo_ref, out_ref,
             comm_ref, send_ref, send_sems, recv_sems):
        my_pos = lax.axis_index("i")
        p1 = my_pos ^ 1
        p2 = (N_DEV - 1) - my_pos

        barrier_sem = pltpu.get_barrier_semaphore()
        for nbr in (p1, p2):
            pl.semaphore_signal(
                barrier_sem, inc=1,
                device_id=(nbr,), device_id_type=pl.DeviceIdType.MESH,
            )
        pl.semaphore_wait(barrier_sem, 2)

        def exchange(slot, partner):
            return pltpu.make_async_remote_copy(
                src_ref=send_ref.at[slot],
                dst_ref=comm_ref.at[slot],
                send_sem=send_sems.at[slot],
                recv_sem=recv_sems.at[slot],
                device_id=(partner,),
                device_id_type=pl.DeviceIdType.MESH,
            )

        x2 = x_ref[...].reshape(B * SQ, D_MODEL).astype(jnp.bfloat16)
        wq = wq_ref[...].astype(jnp.bfloat16)
        wo = wo_ref[...].astype(jnp.bfloat16)
        q = jnp.dot(x2, wq, preferred_element_type=jnp.float32)

        qb = lax.broadcasted_iota(jnp.int32, (SQ, SKV), 0) // BLK
        kb = lax.broadcasted_iota(jnp.int32, (SQ, SKV), 1) // BLK
        mask = kb <= qb

        def batch_partial(b):
            parts = []
            for h in range(HQ_LOC):
                qbh = q[b * SQ:(b + 1) * SQ, h * DH:(h + 1) * DH]
                qbh = qbh.astype(jnp.bfloat16)
                kbh = k_ref[b][:, h * DH:(h + 1) * DH]
                s = lax.dot_general(
                    qbh, kbh, (((1,), (1,)), ((), ())),
                    preferred_element_type=jnp.float32,
                ) * 0.125
                w = jnp.exp(jnp.where(mask, s, -1e9))
                rcp = 1.0 / jnp.sum(w, axis=1, keepdims=True)
                ctx = jnp.dot(w.astype(jnp.bfloat16),
                              v_ref[b][:, h * DH:(h + 1) * DH],
                              preferred_element_type=jnp.float32)
                parts.append((ctx * rcp).astype(jnp.bfloat16))
            ctx_b = jnp.concatenate(parts, axis=1)
            return jnp.dot(ctx_b, wo,
                           preferred_element_type=jnp.float32)

        pA = batch_partial(0)
        send_ref[0] = pA.astype(jnp.bfloat16)
        ex0 = exchange(0, p1)
        ex0.start()

        pB = batch_partial(1)
        send_ref[1] = pB.astype(jnp.bfloat16)
        ex1 = exchange(1, p2)
        ex1.start()

        ex0.wait()
        accA = pA + comm_ref[0].astype(jnp.float32)
        send_ref[2] = accA.astype(jnp.bfloat16)
        ex2 = exchange(2, p2)
        ex2.start()

        ex1.wait()
        accB = pB + comm_ref[1].astype(jnp.float32)
        send_ref[3] = accB.astype(jnp.bfloat16)
        ex3 = exchange(3, p1)
        ex3.start()

        ex2.wait()
        out_ref[0] = accA + comm_ref[2].astype(jnp.float32)
        ex3.wait()
        out_ref[1] = accB + comm_ref[3].astype(jnp.float32)

    return pl.pallas_call(
        body,
        out_shape=jax.ShapeDtypeStruct((B, SQ, D_MODEL), jnp.float32),
        in_specs=[pl.BlockSpec(memory_space=pltpu.VMEM)] * 5,
        out_specs=pl.BlockSpec(memory_space=pltpu.VMEM),
        scratch_shapes=[
            pltpu.VMEM((4, SQ, D_MODEL), jnp.bfloat16),
            pltpu.VMEM((4, SQ, D_MODEL), jnp.bfloat16),
            pltpu.SemaphoreType.DMA((4,)),
            pltpu.SemaphoreType.DMA((4,)),
        ],
        compiler_params=pltpu.CompilerParams(collective_id=0),
    )(x, Wq, k_loc, v_loc, Wo)
